# baseline (speedup 1.0000x reference)
import functools

import jax
import jax.numpy as jnp
from jax import lax
from jax.experimental import pallas as pl
from jax.experimental.pallas import tpu as pltpu
from jax.experimental.pallas import tpu_sc as plsc

B = 16384
VOCAB = 100000
D = 16
NF = 26
ND = 13
K = 8

NC = 2
NS = 16
NW = NC * NS
BPW = B // NW
EPW = BPW * NF
GSZ = 128
NGRP = EPW // GSZ
GPC = 13
CHUNK = GPC * GSZ
NCHUNK = NGRP // GPC


def _sc_gather_body(sp_hbm, table_hbm, out_hbm, sp_v, offs_v, idx_v, rows_v,
                    sem0, sem1):
    wid = lax.axis_index("s") * NC + lax.axis_index("c")
    ebase = wid * EPW

    pltpu.sync_copy(sp_hbm.at[pl.ds(ebase, EPW)], sp_v)

    iota = lax.broadcasted_iota(jnp.int32, (16,), 0)

    def _offs_body(u, _):
        e = iota + u * 16
        offs_v[pl.ds(u * 16, 16)] = (e % NF) * VOCAB
        return 0

    lax.fori_loop(0, CHUNK // 16, _offs_body, 0)

    def build_idx(c, buf):
        def _idx_body(g, _):
            for l in range(GSZ // 16):
                s = sp_v[pl.ds(c * CHUNK + g * GSZ + l * 16, 16)]
                o = offs_v[pl.ds(g * GSZ + l * 16, 16)]
                idx_v[buf, g, pl.ds(l * 16, 16)] = s + o
            return 0
        lax.fori_loop(0, GPC, _idx_body, 0)

    def fire(c, buf, sem):
        cps = []
        for g in range(GPC):
            cps.append(pltpu.async_copy(
                table_hbm.at[idx_v.at[buf, g]],
                rows_v.at[buf, pl.ds(g * GSZ, GSZ)], sem))
        return cps

    def drain(cps, c, buf):
        for cp in cps:
            cp.wait()
        pltpu.sync_copy(rows_v.at[buf],
                        out_hbm.at[pl.ds(ebase + c * CHUNK, CHUNK)])

    sems = (sem0, sem1)
    build_idx(0, 0)
    inflight = fire(0, 0, sems[0])
    for c in range(1, NCHUNK):
        buf = c % 2
        build_idx(c, buf)
        nxt = fire(c, buf, sems[buf])
        drain(inflight, c - 1, (c - 1) % 2)
        inflight = nxt
    drain(inflight, NCHUNK - 1, (NCHUNK - 1) % 2)


def _sc_gather(sp_flat, table_flat):
    mesh = plsc.VectorSubcoreMesh(core_axis_name="c", subcore_axis_name="s")
    return pl.kernel(
        _sc_gather_body,
        out_type=jax.ShapeDtypeStruct((B * NF, D), jnp.float32),
        mesh=mesh,
        compiler_params=pltpu.CompilerParams(use_tc_tiling_on_sc=False),
        scratch_types=[
            pltpu.VMEM((EPW,), jnp.int32),
            pltpu.VMEM((CHUNK,), jnp.int32),
            pltpu.VMEM((2, GPC, GSZ), jnp.int32),
            pltpu.VMEM((2, CHUNK, D), jnp.float32),
            pltpu.SemaphoreType.DMA,
            pltpu.SemaphoreType.DMA,
        ],
    )(sp_flat, table_flat)


def _dense_body(dense_ref, emb_ref, w0_ref, wd_ref, ws_ref, vd_ref, ve_ref,
                w1_ref, b1_ref, w2_ref, b2_ref, w3_ref, b3_ref, w4_ref,
                b4_ref, w5_ref, b5_ref, out_ref):
    f32 = jnp.float32
    dense = dense_ref[...]
    emb = emb_ref[...]

    lin = (jnp.dot(dense, wd_ref[...], preferred_element_type=f32)
           + jnp.dot(emb, ws_ref[...], preferred_element_type=f32))

    vd = vd_ref[...]
    ve = ve_ref[...]
    p = (jnp.dot(dense, vd, preferred_element_type=f32)
         + jnp.dot(emb, ve, preferred_element_type=f32))
    q = (jnp.dot(dense * dense, vd * vd, preferred_element_type=f32)
         + jnp.dot(emb * emb, ve * ve, preferred_element_type=f32))
    inter = 0.5 * jnp.sum(p * p - q, axis=1, keepdims=True)

    h = jnp.maximum(jnp.dot(emb, w1_ref[...], preferred_element_type=f32)
                    + b1_ref[...], 0.0)
    h = jnp.maximum(jnp.dot(h, w2_ref[...], preferred_element_type=f32)
                    + b2_ref[...], 0.0)
    h = jnp.maximum(jnp.dot(h, w3_ref[...], preferred_element_type=f32)
                    + b3_ref[...], 0.0)
    w45 = jnp.dot(w4_ref[...], w5_ref[...], preferred_element_type=f32)
    c45 = jnp.dot(b4_ref[...], w5_ref[...], preferred_element_type=f32) \
        + b5_ref[...]
    deep = jnp.dot(h, w45, preferred_element_type=f32) + c45

    logit = 0.5 * (lin + w0_ref[0, 0] + inter + deep)
    out_ref[...] = 1.0 / (1.0 + jnp.exp(-logit))


def _dense_stage(dense, emb, w0, wd, ws, vd, ve, W1, b1, W2, b2, W3, b3,
                 W4, b4, W5, b5):
    BB = 1024
    grid = (B // BB,)

    def batch_spec(cols):
        return pl.BlockSpec((BB, cols), lambda i: (i, 0))

    def full_spec(a):
        return pl.BlockSpec(a.shape, lambda i: (0,) * a.ndim)

    return pl.pallas_call(
        _dense_body,
        grid=grid,
        in_specs=[
            batch_spec(ND), batch_spec(NF * D),
            full_spec(w0), full_spec(wd), full_spec(ws), full_spec(vd),
            full_spec(ve), full_spec(W1), full_spec(b1), full_spec(W2),
            full_spec(b2), full_spec(W3), full_spec(b3), full_spec(W4),
            full_spec(b4), full_spec(W5), full_spec(b5),
        ],
        out_specs=batch_spec(1),
        out_shape=jax.ShapeDtypeStruct((B, 1), jnp.float32),
    )(dense, emb, w0, wd, ws, vd, ve, W1, b1, W2, b2, W3, b3, W4, b4, W5, b5)


def kernel(inputs, tables, w0, w, V, W1, b1, W2, b2, W3, b3, W4, b4, W5, b5):
    sp_flat = inputs[:, ND:].reshape(B * NF)
    table_flat = tables.reshape(NF * VOCAB, D)
    emb = _sc_gather(sp_flat, table_flat).reshape(B, NF * D)

    dense = inputs[:, :ND].astype(jnp.float32)
    return _dense_stage(
        dense, emb, w0.reshape(1, 1), w[:ND], w[ND:], V[:ND], V[ND:],
        W1, b1.reshape(1, 256), W2, b2.reshape(1, 128), W3, b3.reshape(1, 64),
        W4, b4.reshape(1, 64), W5, b5.reshape(1, 1))
